# baseline (speedup 1.0000x reference)
import functools

import jax
import jax.numpy as jnp
from jax import lax
from jax.experimental import pallas as pl
from jax.experimental.pallas import tpu as pltpu
from jax.experimental.pallas import tpu_sc as plsc

N = 10000
D = 128
E = 320000
NC, NS = 2, 16
NW = NC * NS
K = 80
NCHUNK = 4096
EPAD = NCHUNK * K
NPAD = 10240
RPT = NPAD // NS
GSZ = 16
NB = 4
NG0, NG1 = 16, 0
CPW0, CPW1 = NG0 * GSZ, NG1 * GSZ
DEG_K = 128
DEG_CPW = 80

_mesh = plsc.VectorSubcoreMesh(core_axis_name="c", subcore_axis_name="s")


def _fill2d(ref, rows, cols, value):
    v = jnp.full((16,), value, jnp.float32)
    per_row = cols // 16

    def body(i, carry):
        r = i // per_row
        c = (i % per_row) * 16
        ref[r, pl.ds(c, 16)] = v
        return carry

    lax.fori_loop(0, rows * per_row, body, 0)


def _fill1d(ref, n, value):
    v = jnp.full((16,), value, jnp.float32)

    def body(i, carry):
        ref[pl.ds(i * 16, 16)] = v
        return carry

    lax.fori_loop(0, n // 16, body, 0)


def _deg_body(dst2d, out, dst_v, stage_v, ones_v, deg_sh):
    cid = lax.axis_index("c")
    sid = lax.axis_index("s")
    wid = sid * NC + cid
    _fill1d(stage_v, RPT, 0.0)
    pltpu.sync_copy(stage_v, deg_sh.at[pl.ds(sid * RPT, RPT)])
    _fill1d(ones_v, DEG_K, 1.0)
    pltpu.sync_copy(dst2d.at[pl.ds(wid * DEG_CPW, DEG_CPW)], dst_v)
    plsc.subcore_barrier()

    def body(j, carry):
        pltpu.sync_copy(ones_v, deg_sh.at[dst_v.at[j]], add=True)
        return carry

    lax.fori_loop(0, DEG_CPW, body, 0)
    plsc.subcore_barrier()
    pltpu.sync_copy(deg_sh.at[pl.ds(sid * RPT, RPT)], stage_v)
    pltpu.sync_copy(stage_v, out.at[pl.ds(cid * NPAD + sid * RPT, RPT)])


_deg_kernel = functools.partial(
    pl.kernel,
    out_type=jax.ShapeDtypeStruct((NC * NPAD,), jnp.float32),
    mesh=_mesh,
    scratch_types=[
        pltpu.VMEM((DEG_CPW, DEG_K), jnp.int32),
        pltpu.VMEM((RPT,), jnp.float32),
        pltpu.VMEM((DEG_K,), jnp.float32),
        pltpu.VMEM_SHARED((NPAD,), jnp.float32),
    ],
)(_deg_body)


def _zero_buf0(rows_v):
    v = jnp.zeros((16,), jnp.float32)

    def body(i, carry):
        r = i // (D // 16)
        c = (i % (D // 16)) * 16
        rows_v[0, r, pl.ds(c, 16)] = v
        return carry

    lax.fori_loop(0, K * (D // 16), body, 0)


def _scat_body(hs, src2d, dst2d, out, src_v, dst_v, rows_v, sem_g, sem_s, acc_sh):
    cid = lax.axis_index("c")
    sid = lax.axis_index("s")
    wid = sid * NC + cid
    _zero_buf0(rows_v)
    for t in range(RPT // K):
        pltpu.sync_copy(rows_v.at[0], acc_sh.at[pl.ds(sid * RPT + t * K, K)])
    plsc.subcore_barrier()

    my_ng = jnp.where(cid == 0, NG0, NG1)
    my_base = jnp.where(cid == 0, sid * CPW0, NS * CPW0 + sid * CPW1)

    def group(g, carry):
        base = my_base + g * GSZ
        pltpu.sync_copy(src2d.at[pl.ds(base, GSZ)], src_v)
        pltpu.sync_copy(dst2d.at[pl.ds(base, GSZ)], dst_v)
        pltpu.async_copy(hs.at[src_v.at[0]], rows_v.at[0], sem_g)
        pltpu.async_copy(hs.at[src_v.at[1]], rows_v.at[1], sem_g)

        def body(j, carry):
            b = lax.rem(j, NB)

            @pl.when(j >= 2)
            def _drain():
                b2 = lax.rem(j - 2, NB)
                pltpu.make_async_copy(
                    rows_v.at[b2], acc_sh.at[dst_v.at[j - 2]], sem_s
                ).wait()

            @pl.when(j + 2 < GSZ)
            def _prefetch():
                b3 = lax.rem(j + 2, NB)
                pltpu.async_copy(hs.at[src_v.at[j + 2]], rows_v.at[b3], sem_g)

            pltpu.make_async_copy(hs.at[src_v.at[j]], rows_v.at[b], sem_g).wait()
            pltpu.async_copy(rows_v.at[b], acc_sh.at[dst_v.at[j]], sem_s, add=True)
            return carry

        lax.fori_loop(0, GSZ, body, 0)
        for j in (GSZ - 2, GSZ - 1):
            pltpu.make_async_copy(
                rows_v.at[j % NB], acc_sh.at[dst_v.at[j]], sem_s
            ).wait()
        return carry

    lax.fori_loop(0, my_ng, group, 0)
    plsc.subcore_barrier()
    for t in range(RPT // K):
        pltpu.sync_copy(acc_sh.at[pl.ds(sid * RPT + t * K, K)], rows_v.at[0])
        pltpu.sync_copy(rows_v.at[0], out.at[pl.ds(cid * NPAD + sid * RPT + t * K, K)])


_scat_kernel = functools.partial(
    pl.kernel,
    out_type=jax.ShapeDtypeStruct((NC * NPAD, D), jnp.float32),
    mesh=_mesh,
    scratch_types=[
        pltpu.VMEM((GSZ, K), jnp.int32),
        pltpu.VMEM((GSZ, K), jnp.int32),
        pltpu.VMEM((NB, K, D), jnp.float32),
        pltpu.SemaphoreType.DMA,
        pltpu.SemaphoreType.DMA,
        pltpu.VMEM_SHARED((NPAD, D), jnp.float32),
    ],
)(_scat_body)


def _mm_body(x_ref, w_ref, deg_ref, hs_ref):
    dinv = lax.rsqrt(deg_ref[...])
    h = jnp.dot(x_ref[...], w_ref[...], preferred_element_type=jnp.float32)
    hs_ref[...] = h * dinv


def _fin_body(acc_ref, hs_ref, deg_ref, b_ref, gam_ref, bet_ref, out_ref):
    dinv = lax.rsqrt(deg_ref[...])
    t = (acc_ref[0:N, :] + acc_ref[NPAD : NPAD + N, :] + hs_ref[...]) * dinv + b_ref[...]
    mean = jnp.mean(t, axis=0, keepdims=True)
    var = jnp.mean((t - mean) ** 2, axis=0, keepdims=True)
    out_ref[...] = jnp.maximum(
        (t - mean) * lax.rsqrt(var + 1e-5) * gam_ref[...] + bet_ref[...], 0.0
    )


def kernel(x, edge_index, W, b, gamma, beta):
    ei = edge_index.astype(jnp.int32)
    pad = EPAD - E
    src2d = jnp.concatenate([ei[0], jnp.zeros((pad,), jnp.int32)]).reshape(NCHUNK, K)
    dst2d = jnp.concatenate([ei[1], jnp.full((pad,), N, jnp.int32)]).reshape(NCHUNK, K)

    degp = _deg_kernel(dst2d.reshape(NW * DEG_CPW, DEG_K))
    deg_col = (degp[0:N] + degp[NPAD : NPAD + N] + 1.0)[:, None]

    hs = pl.pallas_call(
        _mm_body,
        out_shape=jax.ShapeDtypeStruct((N, D), jnp.float32),
    )(x, W, deg_col)

    accp = _scat_kernel(hs, src2d, dst2d)

    out = pl.pallas_call(
        _fin_body,
        out_shape=jax.ShapeDtypeStruct((N, D), jnp.float32),
    )(
        accp,
        hs,
        deg_col,
        b.reshape(1, D),
        gamma.reshape(1, D),
        beta.reshape(1, D),
    )
    return out


# speedup vs baseline: 1.2629x; 1.2629x over previous
import functools

import jax
import jax.numpy as jnp
from jax import lax
from jax.experimental import pallas as pl
from jax.experimental.pallas import tpu as pltpu
from jax.experimental.pallas import tpu_sc as plsc

N = 10000
D = 128
E = 320000
NC, NS = 2, 16
NW = NC * NS
K = 128
NCHUNK = 2560
EPAD = NCHUNK * K
NPAD = 10240
RPT = NPAD // NS
GSZ = 16
RSTEP = 64
NB = 2
NG0, NG1 = 8, 2
CPW0, CPW1 = NG0 * GSZ, NG1 * GSZ
DEG_K = 128
DEG_CPW = 80

_mesh = plsc.VectorSubcoreMesh(core_axis_name="c", subcore_axis_name="s")


def _fill2d(ref, rows, cols, value):
    v = jnp.full((16,), value, jnp.float32)
    per_row = cols // 16

    def body(i, carry):
        r = i // per_row
        c = (i % per_row) * 16
        ref[r, pl.ds(c, 16)] = v
        return carry

    lax.fori_loop(0, rows * per_row, body, 0)


def _fill1d(ref, n, value):
    v = jnp.full((16,), value, jnp.float32)

    def body(i, carry):
        ref[pl.ds(i * 16, 16)] = v
        return carry

    lax.fori_loop(0, n // 16, body, 0)


def _deg_body(dst2d, out, dst_v, stage_v, ones_v, deg_sh):
    cid = lax.axis_index("c")
    sid = lax.axis_index("s")
    wid = sid * NC + cid
    _fill1d(stage_v, RPT, 0.0)
    pltpu.sync_copy(stage_v, deg_sh.at[pl.ds(sid * RPT, RPT)])
    _fill1d(ones_v, DEG_K, 1.0)
    pltpu.sync_copy(dst2d.at[pl.ds(wid * DEG_CPW, DEG_CPW)], dst_v)
    plsc.subcore_barrier()

    def body(j, carry):
        pltpu.sync_copy(ones_v, deg_sh.at[dst_v.at[j]], add=True)
        return carry

    lax.fori_loop(0, DEG_CPW, body, 0)
    plsc.subcore_barrier()
    pltpu.sync_copy(deg_sh.at[pl.ds(sid * RPT, RPT)], stage_v)
    pltpu.sync_copy(stage_v, out.at[pl.ds(cid * NPAD + sid * RPT, RPT)])


_deg_kernel = functools.partial(
    pl.kernel,
    out_type=jax.ShapeDtypeStruct((NC * NPAD,), jnp.float32),
    mesh=_mesh,
    scratch_types=[
        pltpu.VMEM((DEG_CPW, DEG_K), jnp.int32),
        pltpu.VMEM((RPT,), jnp.float32),
        pltpu.VMEM((DEG_K,), jnp.float32),
        pltpu.VMEM_SHARED((NPAD,), jnp.float32),
    ],
)(_deg_body)


def _zero_buf0(rows_v):
    v = jnp.zeros((16,), jnp.float32)

    def body(i, carry):
        r = i // (D // 16)
        c = (i % (D // 16)) * 16
        rows_v[0, r, pl.ds(c, 16)] = v
        return carry

    lax.fori_loop(0, RSTEP * (D // 16), body, 0)


def _scat_body(hs, src2d, dst2d, out, src_v, dst_v, rows_v, sem_g, sem_s, acc_sh):
    cid = lax.axis_index("c")
    sid = lax.axis_index("s")
    wid = sid * NC + cid
    _zero_buf0(rows_v)
    for t in range(RPT // RSTEP):
        pltpu.sync_copy(rows_v.at[0, pl.ds(0, RSTEP)],
                        acc_sh.at[pl.ds(sid * RPT + t * RSTEP, RSTEP)])
    plsc.subcore_barrier()

    def group(base):
        pltpu.sync_copy(src2d.at[pl.ds(base, GSZ)], src_v)
        pltpu.sync_copy(dst2d.at[pl.ds(base, GSZ)], dst_v)
        pltpu.async_copy(hs.at[src_v.at[0]], rows_v.at[0], sem_g)

        def body(j, carry):
            b = lax.rem(j, NB)
            pltpu.make_async_copy(hs.at[src_v.at[j]], rows_v.at[b], sem_g).wait()

            @pl.when(j < GSZ - 1)
            def _prefetch():
                pltpu.async_copy(hs.at[src_v.at[j + 1]], rows_v.at[1 - b], sem_g)

            pltpu.sync_copy(rows_v.at[b], acc_sh.at[dst_v.at[j]], add=True)
            return carry

        lax.fori_loop(0, GSZ, body, 0)

    @pl.when(cid == 0)
    def _sc0():
        for g in range(NG0):
            group(sid * CPW0 + g * GSZ)

    @pl.when(cid == 1)
    def _sc1():
        for g in range(NG1):
            group(NS * CPW0 + sid * CPW1 + g * GSZ)

    plsc.subcore_barrier()
    for t in range(RPT // RSTEP):
        pltpu.sync_copy(acc_sh.at[pl.ds(sid * RPT + t * RSTEP, RSTEP)],
                        rows_v.at[0, pl.ds(0, RSTEP)])
        pltpu.sync_copy(rows_v.at[0, pl.ds(0, RSTEP)],
                        out.at[pl.ds(cid * NPAD + sid * RPT + t * RSTEP, RSTEP)])


_scat_kernel = functools.partial(
    pl.kernel,
    out_type=jax.ShapeDtypeStruct((NC * NPAD, D), jnp.float32),
    mesh=_mesh,
    scratch_types=[
        pltpu.VMEM((GSZ, K), jnp.int32),
        pltpu.VMEM((GSZ, K), jnp.int32),
        pltpu.VMEM((NB, K, D), jnp.float32),
        pltpu.SemaphoreType.DMA,
        pltpu.SemaphoreType.DMA,
        pltpu.VMEM_SHARED((NPAD, D), jnp.float32),
    ],
)(_scat_body)


def _mm_body(x_ref, w_ref, deg_ref, hs_ref):
    dinv = lax.rsqrt(deg_ref[...])
    h = jnp.dot(x_ref[...], w_ref[...], preferred_element_type=jnp.float32)
    hs_ref[...] = h * dinv


def _fin_body(acc_ref, hs_ref, deg_ref, b_ref, gam_ref, bet_ref, out_ref):
    dinv = lax.rsqrt(deg_ref[...])
    t = (acc_ref[0:N, :] + acc_ref[NPAD : NPAD + N, :] + hs_ref[...]) * dinv + b_ref[...]
    mean = jnp.mean(t, axis=0, keepdims=True)
    var = jnp.mean((t - mean) ** 2, axis=0, keepdims=True)
    out_ref[...] = jnp.maximum(
        (t - mean) * lax.rsqrt(var + 1e-5) * gam_ref[...] + bet_ref[...], 0.0
    )


def kernel(x, edge_index, W, b, gamma, beta):
    ei = edge_index.astype(jnp.int32)
    pad = EPAD - E
    src2d = jnp.concatenate([ei[0], jnp.zeros((pad,), jnp.int32)]).reshape(NCHUNK, K)
    dst2d = jnp.concatenate([ei[1], jnp.full((pad,), N, jnp.int32)]).reshape(NCHUNK, K)

    degp = _deg_kernel(dst2d.reshape(NW * DEG_CPW, DEG_K))
    deg_col = (degp[0:N] + degp[NPAD : NPAD + N] + 1.0)[:, None]

    hs = pl.pallas_call(
        _mm_body,
        out_shape=jax.ShapeDtypeStruct((N, D), jnp.float32),
    )(x, W, deg_col)

    accp = _scat_kernel(hs, src2d, dst2d)

    out = pl.pallas_call(
        _fin_body,
        out_shape=jax.ShapeDtypeStruct((N, D), jnp.float32),
    )(
        accp,
        hs,
        deg_col,
        b.reshape(1, D),
        gamma.reshape(1, D),
        beta.reshape(1, D),
    )
    return out
